# baseline (speedup 1.0000x reference)
import functools

import jax
import jax.numpy as jnp
from jax import lax
from jax.experimental import pallas as pl
from jax.experimental.pallas import tpu as pltpu
from jax.experimental.pallas import tpu_sc as plsc

NC = 2
NS = 16
NW = NC * NS
TZ = 128
CAP = 98304
CHUNK = 8192
NCHUNK = CAP // CHUNK
LANES = 16
NOUT = 192


def _sc_body(f1t_hbm, f2t_hbm, i_hbm, j_hbm, o_hbm, v_hbm, out_hbm,
             f1_v, f2_v, out_v, i_v, j_v, o_v, v_v):
    wid = lax.axis_index("s") * NC + lax.axis_index("c")
    zbase = wid * TZ
    pltpu.sync_copy(f1t_hbm.at[:, pl.ds(zbase, TZ)], f1_v)
    pltpu.sync_copy(f2t_hbm.at[:, pl.ds(zbase, TZ)], f2_v)

    def zinit(t, carry):
        for r in range(TZ // LANES):
            out_v[t, pl.ds(r * LANES, LANES)] = jnp.zeros((LANES,), jnp.float32)
        return carry

    lax.fori_loop(0, NOUT + 1, zinit, 0)

    def chunk_body(c, carry):
        base = c * CHUNK
        pltpu.sync_copy(i_hbm.at[pl.ds(base, CHUNK)], i_v)
        pltpu.sync_copy(j_hbm.at[pl.ds(base, CHUNK)], j_v)
        pltpu.sync_copy(o_hbm.at[pl.ds(base, CHUNK)], o_v)
        pltpu.sync_copy(v_hbm.at[pl.ds(base, CHUNK)], v_v)

        def group_body(g, inner):
            gsl = pl.ds(g * LANES, LANES)
            iv = i_v[gsl]
            jv = j_v[gsl]
            ov = o_v[gsl]
            vv = v_v[gsl]
            for t in range(LANES):
                ii = iv[t]
                jj = jv[t]
                oo = ov[t]
                vt = vv[t]
                for r in range(TZ // LANES):
                    sl = pl.ds(r * LANES, LANES)
                    prod = f1_v[ii, sl] * f2_v[jj, sl]
                    plsc.addupdate(out_v.at[oo, sl], prod * vt)
            return inner

        return lax.fori_loop(0, CHUNK // LANES, group_body, carry)

    lax.fori_loop(0, NCHUNK, chunk_body, 0)
    pltpu.sync_copy(out_v.at[pl.ds(0, 192)], out_hbm.at[:, pl.ds(zbase, TZ)])


def kernel(features_1, features_2, mixing_matrix):
    z, n1 = features_1.shape
    n2 = features_2.shape[1]
    n_out = mixing_matrix.shape[0]
    f1t = features_1.T
    f2t = features_2.T
    flat = mixing_matrix.reshape(-1)
    nzmask = flat != 0.0
    count = jnp.count_nonzero(nzmask)
    idx = jnp.nonzero(nzmask, size=CAP, fill_value=0)[0].astype(jnp.int32)
    valid = jnp.arange(CAP, dtype=jnp.int32) < count
    vals = jnp.where(valid, flat[idx], 0.0)
    o_id = idx // (n1 * n2)
    k_id = idx % (n1 * n2)
    i_id = k_id // n2
    j_id = k_id % n2
    o_id = jnp.where(valid, o_id, n_out)

    sc = pl.kernel(
        _sc_body,
        out_type=jax.ShapeDtypeStruct((n_out, z), jnp.float32),
        mesh=plsc.VectorSubcoreMesh(
            core_axis_name="c", subcore_axis_name="s",
            num_cores=NC, num_subcores=NS),
        scratch_types=[
            pltpu.VMEM((n1, TZ), jnp.float32),
            pltpu.VMEM((n2, TZ), jnp.float32),
            pltpu.VMEM((n_out + 1, TZ), jnp.float32),
            pltpu.VMEM((CHUNK,), jnp.int32),
            pltpu.VMEM((CHUNK,), jnp.int32),
            pltpu.VMEM((CHUNK,), jnp.int32),
            pltpu.VMEM((CHUNK,), jnp.float32),
        ],
    )
    outt = sc(f1t, f2t, i_id, j_id, o_id, vals)
    return outt.T
